# baseline (speedup 1.0000x reference)
import functools

import jax
import jax.numpy as jnp
from jax import lax
from jax.experimental import pallas as pl
from jax.experimental.pallas import tpu as pltpu
from jax.experimental.pallas import tpu_sc as plsc

N = 10000
E = 320000
D = 128
G = 64

NC = 2
NS = 16
NW = NC * NS
CHUNK = 128
NCH = 79
EW = NCH * CHUNK
E_PAD = NW * EW
N_PAD = 10240
RPT = N_PAD // NS
DUMP = N

BLK = 512
NBLK = N_PAD // BLK

_mesh = plsc.VectorSubcoreMesh(core_axis_name="c", subcore_axis_name="s")


@functools.partial(
    pl.kernel,
    out_type=jax.ShapeDtypeStruct((NC, N_PAD, D), jnp.float32),
    mesh=_mesh,
    scratch_types=[
        pltpu.VMEM((NCH, CHUNK), jnp.int32),
        pltpu.VMEM((NCH, CHUNK), jnp.int32),
        pltpu.VMEM((CHUNK, D), jnp.float32),
        pltpu.VMEM_SHARED((N_PAD, D), jnp.float32),
    ],
)
def _sc_agg(src_hbm, dst_hbm, h_hbm, zeros_hbm, out_hbm, src_v, dst_v, rows_v,
            agg_sh):
    cid = lax.axis_index("c")
    sid = lax.axis_index("s")
    wid = sid * NC + cid

    pltpu.sync_copy(zeros_hbm, agg_sh.at[pl.ds(sid * RPT, RPT)])
    pltpu.sync_copy(src_hbm.at[wid], src_v)
    pltpu.sync_copy(dst_hbm.at[wid], dst_v)

    plsc.subcore_barrier()

    def body(j, carry):
        pltpu.sync_copy(h_hbm.at[src_v.at[j]], rows_v)
        pltpu.sync_copy(rows_v, agg_sh.at[dst_v.at[j]], add=True)
        return carry

    lax.fori_loop(0, NCH, body, 0)

    plsc.subcore_barrier()

    pltpu.sync_copy(agg_sh.at[pl.ds(sid * RPT, RPT)],
                    out_hbm.at[cid, pl.ds(sid * RPT, RPT)])


def _tc_layer_body(h_ref, a0_ref, a1_ref, w_ref, b_ref, o_ref):
    s = h_ref[...] + a0_ref[...] + a1_ref[...]
    o_ref[...] = jnp.maximum(
        jnp.dot(s, w_ref[...], preferred_element_type=jnp.float32) + b_ref[...],
        0.0)


_tc_layer = pl.pallas_call(
    _tc_layer_body,
    grid=(NBLK,),
    in_specs=[
        pl.BlockSpec((BLK, D), lambda i: (i, 0)),
        pl.BlockSpec((BLK, D), lambda i: (i, 0)),
        pl.BlockSpec((BLK, D), lambda i: (i, 0)),
        pl.BlockSpec((D, D), lambda i: (0, 0)),
        pl.BlockSpec((1, D), lambda i: (0, 0)),
    ],
    out_specs=pl.BlockSpec((BLK, D), lambda i: (i, 0)),
    out_shape=jax.ShapeDtypeStruct((N_PAD, D), jnp.float32),
)


def _tc_final_body(h_ref, a0_ref, a1_ref, w_ref, b_ref, bi_ref, wp_ref,
                   bp_ref, o_ref, sums, counts):
    i = pl.program_id(0)

    @pl.when(i == 0)
    def _():
        sums[...] = jnp.zeros_like(sums)
        counts[...] = jnp.zeros_like(counts)

    s = h_ref[...] + a0_ref[...] + a1_ref[...]
    h3 = jnp.maximum(
        jnp.dot(s, w_ref[...], preferred_element_type=jnp.float32) + b_ref[...],
        0.0)

    bi = bi_ref[0, 0]
    onehot = (bi[:, None] == lax.broadcasted_iota(jnp.int32, (1, G), 1)
              ).astype(jnp.float32)
    sums[...] += lax.dot_general(onehot, h3, (((0,), (0,)), ((), ())),
                                 preferred_element_type=jnp.float32)
    counts[...] += jnp.broadcast_to(jnp.sum(onehot, axis=0)[:, None], (G, D))

    @pl.when(i == NBLK - 1)
    def _():
        pooled = sums[...] / jnp.maximum(counts[...], 1.0)
        o_ref[...] = jnp.maximum(
            jnp.dot(pooled, wp_ref[...], preferred_element_type=jnp.float32)
            + bp_ref[...], 0.0)


_tc_final = pl.pallas_call(
    _tc_final_body,
    grid=(NBLK,),
    in_specs=[
        pl.BlockSpec((BLK, D), lambda i: (i, 0)),
        pl.BlockSpec((BLK, D), lambda i: (i, 0)),
        pl.BlockSpec((BLK, D), lambda i: (i, 0)),
        pl.BlockSpec((D, D), lambda i: (0, 0)),
        pl.BlockSpec((1, D), lambda i: (0, 0)),
        pl.BlockSpec((1, 1, BLK), lambda i: (i, 0, 0)),
        pl.BlockSpec((D, D), lambda i: (0, 0)),
        pl.BlockSpec((1, D), lambda i: (0, 0)),
    ],
    out_specs=pl.BlockSpec((G, D), lambda i: (0, 0)),
    out_shape=jax.ShapeDtypeStruct((G, D), jnp.float32),
    scratch_shapes=[
        pltpu.VMEM((G, D), jnp.float32),
        pltpu.VMEM((G, D), jnp.float32),
    ],
)


def kernel(x, edge_index, batch_index, W0, b0, W1, b1, W2, b2, Wp, bp):
    src = edge_index[0]
    dst = edge_index[1]
    pad_e = E_PAD - E
    src_r = jnp.concatenate(
        [src, jnp.zeros((pad_e,), jnp.int32)]).reshape(NW, NCH, CHUNK)
    pad_dst = N + (jnp.arange(pad_e, dtype=jnp.int32) % (N_PAD - N))
    dst_r = jnp.concatenate([dst, pad_dst]).reshape(NW, NCH, CHUNK)
    bi_r = jnp.pad(batch_index, (0, N_PAD - N),
                   constant_values=G).reshape(NBLK, 1, BLK)
    zeros_tile = jnp.zeros((RPT, D), jnp.float32)

    h = jnp.pad(x, ((0, N_PAD - N), (0, 0)))
    for (W, b) in ((W0, b0), (W1, b1)):
        agg = _sc_agg(src_r, dst_r, h, zeros_tile)
        h = _tc_layer(h, agg[0], agg[1], W, b.reshape(1, D))

    agg = _sc_agg(src_r, dst_r, h, zeros_tile)
    return _tc_final(h, agg[0], agg[1], W2, b2.reshape(1, D), bi_r, Wp,
                     bp.reshape(1, D))


# speedup vs baseline: 1.0296x; 1.0296x over previous
import functools

import jax
import jax.numpy as jnp
from jax import lax
from jax.experimental import pallas as pl
from jax.experimental.pallas import tpu as pltpu
from jax.experimental.pallas import tpu_sc as plsc

N = 10000
E = 320000
D = 128
G = 64

NC = 2
NS = 16
NW = NC * NS
CHUNK = 128
NCH = 79
NCH_A = 102
NCH_B = 2 * NCH - NCH_A
EW = NCH * CHUNK
E_PAD = NS * (NCH_A + NCH_B) * CHUNK
N_PAD = 10240
RPT = N_PAD // NS

BLK = 512
NBLK = N_PAD // BLK

_mesh = plsc.VectorSubcoreMesh(core_axis_name="c", subcore_axis_name="s")


@functools.partial(
    pl.kernel,
    out_type=jax.ShapeDtypeStruct((NC, N_PAD, D), jnp.float32),
    mesh=_mesh,
    scratch_types=[
        pltpu.VMEM((NCH_A, CHUNK), jnp.int32),
        pltpu.VMEM((NCH_A, CHUNK), jnp.int32),
        pltpu.VMEM((CHUNK, D), jnp.float32),
        pltpu.VMEM_SHARED((N_PAD, D), jnp.float32),
    ],
)
def _sc_agg(src_a_hbm, dst_a_hbm, src_b_hbm, dst_b_hbm, h_hbm, zeros_hbm,
            out_hbm, src_v, dst_v, rows_v, agg_sh):
    cid = lax.axis_index("c")
    sid = lax.axis_index("s")

    pltpu.sync_copy(zeros_hbm, agg_sh.at[pl.ds(sid * RPT, RPT)])

    @pl.when(cid == 0)
    def _():
        pltpu.sync_copy(src_a_hbm.at[sid], src_v)
        pltpu.sync_copy(dst_a_hbm.at[sid], dst_v)

    @pl.when(cid == 1)
    def _():
        pltpu.sync_copy(src_b_hbm.at[sid], src_v.at[pl.ds(0, NCH_B)])
        pltpu.sync_copy(dst_b_hbm.at[sid], dst_v.at[pl.ds(0, NCH_B)])

    plsc.subcore_barrier()

    nch = jnp.where(cid == 0, NCH_A, NCH_B)

    def body(j, carry):
        pltpu.sync_copy(h_hbm.at[src_v.at[j]], rows_v)
        pltpu.sync_copy(rows_v, agg_sh.at[dst_v.at[j]], add=True)
        return carry

    lax.fori_loop(0, nch, body, 0)

    plsc.subcore_barrier()

    pltpu.sync_copy(agg_sh.at[pl.ds(sid * RPT, RPT)],
                    out_hbm.at[cid, pl.ds(sid * RPT, RPT)])


def _tc_layer_body(h_ref, a0_ref, a1_ref, w_ref, b_ref, o_ref):
    s = h_ref[...] + a0_ref[...] + a1_ref[...]
    o_ref[...] = jnp.maximum(
        jnp.dot(s, w_ref[...], preferred_element_type=jnp.float32) + b_ref[...],
        0.0)


_tc_layer = pl.pallas_call(
    _tc_layer_body,
    grid=(NBLK,),
    in_specs=[
        pl.BlockSpec((BLK, D), lambda i: (i, 0)),
        pl.BlockSpec((BLK, D), lambda i: (i, 0)),
        pl.BlockSpec((BLK, D), lambda i: (i, 0)),
        pl.BlockSpec((D, D), lambda i: (0, 0)),
        pl.BlockSpec((1, D), lambda i: (0, 0)),
    ],
    out_specs=pl.BlockSpec((BLK, D), lambda i: (i, 0)),
    out_shape=jax.ShapeDtypeStruct((N_PAD, D), jnp.float32),
)


def _tc_final_body(h_ref, a0_ref, a1_ref, w_ref, b_ref, bi_ref, wp_ref,
                   bp_ref, o_ref, sums, counts):
    i = pl.program_id(0)

    @pl.when(i == 0)
    def _():
        sums[...] = jnp.zeros_like(sums)
        counts[...] = jnp.zeros_like(counts)

    s = h_ref[...] + a0_ref[...] + a1_ref[...]
    h3 = jnp.maximum(
        jnp.dot(s, w_ref[...], preferred_element_type=jnp.float32) + b_ref[...],
        0.0)

    bi = bi_ref[0, 0]
    onehot = (bi[:, None] == lax.broadcasted_iota(jnp.int32, (1, G), 1)
              ).astype(jnp.float32)
    sums[...] += lax.dot_general(onehot, h3, (((0,), (0,)), ((), ())),
                                 preferred_element_type=jnp.float32)
    counts[...] += jnp.broadcast_to(jnp.sum(onehot, axis=0)[:, None], (G, D))

    @pl.when(i == NBLK - 1)
    def _():
        pooled = sums[...] / jnp.maximum(counts[...], 1.0)
        o_ref[...] = jnp.maximum(
            jnp.dot(pooled, wp_ref[...], preferred_element_type=jnp.float32)
            + bp_ref[...], 0.0)


_tc_final = pl.pallas_call(
    _tc_final_body,
    grid=(NBLK,),
    in_specs=[
        pl.BlockSpec((BLK, D), lambda i: (i, 0)),
        pl.BlockSpec((BLK, D), lambda i: (i, 0)),
        pl.BlockSpec((BLK, D), lambda i: (i, 0)),
        pl.BlockSpec((D, D), lambda i: (0, 0)),
        pl.BlockSpec((1, D), lambda i: (0, 0)),
        pl.BlockSpec((1, 1, BLK), lambda i: (i, 0, 0)),
        pl.BlockSpec((D, D), lambda i: (0, 0)),
        pl.BlockSpec((1, D), lambda i: (0, 0)),
    ],
    out_specs=pl.BlockSpec((G, D), lambda i: (0, 0)),
    out_shape=jax.ShapeDtypeStruct((G, D), jnp.float32),
    scratch_shapes=[
        pltpu.VMEM((G, D), jnp.float32),
        pltpu.VMEM((G, D), jnp.float32),
    ],
)


def kernel(x, edge_index, batch_index, W0, b0, W1, b1, W2, b2, Wp, bp):
    src = edge_index[0]
    dst = edge_index[1]
    pad_e = E_PAD - E
    src_p = jnp.concatenate([src, jnp.zeros((pad_e,), jnp.int32)])
    pad_dst = N + (jnp.arange(pad_e, dtype=jnp.int32) % (N_PAD - N))
    dst_p = jnp.concatenate([dst, pad_dst])
    ea = NS * NCH_A * CHUNK
    src_a = src_p[:ea].reshape(NS, NCH_A, CHUNK)
    dst_a = dst_p[:ea].reshape(NS, NCH_A, CHUNK)
    src_b = src_p[ea:].reshape(NS, NCH_B, CHUNK)
    dst_b = dst_p[ea:].reshape(NS, NCH_B, CHUNK)
    bi_r = jnp.pad(batch_index, (0, N_PAD - N),
                   constant_values=G).reshape(NBLK, 1, BLK)
    zeros_tile = jnp.zeros((RPT, D), jnp.float32)

    h = jnp.pad(x, ((0, N_PAD - N), (0, 0)))
    for (W, b) in ((W0, b0), (W1, b1)):
        agg = _sc_agg(src_a, dst_a, src_b, dst_b, h, zeros_tile)
        h = _tc_layer(h, agg[0], agg[1], W, b.reshape(1, D))

    agg = _sc_agg(src_a, dst_a, src_b, dst_b, h, zeros_tile)
    return _tc_final(h, agg[0], agg[1], W2, b2.reshape(1, D), bi_r, Wp,
                     bp.reshape(1, D))
